# baseline (speedup 1.0000x reference)
import functools

import jax
import jax.numpy as jnp
from jax import lax
from jax.experimental import pallas as pl
from jax.experimental.pallas import tpu as pltpu
from jax.experimental.pallas import tpu_sc as plsc

_BT = 256
_NC = 2
_NS = 16
_NW = _NC * _NS


def _router_body(x_ref, wg_ref, bg_ref, wn_ref, bn_ref, eps_ref, g_ref):
    xb = x_ref[...]
    lg = jnp.dot(xb, wg_ref[...], preferred_element_type=jnp.float32) + bg_ref[...]
    nz = jnp.dot(xb, wn_ref[...], preferred_element_type=jnp.float32) + bn_ref[...]
    sp = jnp.maximum(nz, 0.0) + jnp.log1p(jnp.exp(-jnp.abs(nz)))
    nl = lg + eps_ref[...] * sp
    e = nl.shape[-1]
    m1 = jnp.max(nl, axis=-1, keepdims=True)
    ii = jax.lax.broadcasted_iota(jnp.int32, nl.shape, 1)
    fmi = jnp.min(jnp.where(nl == m1, ii, e), axis=-1, keepdims=True)
    m2 = jnp.max(jnp.where(ii == fmi, -jnp.inf, nl), axis=-1, keepdims=True)
    sel = (ii == fmi) | (nl >= m2)
    z = jnp.where(sel, jnp.exp(nl - m1), 0.0)
    g_ref[...] = z / jnp.sum(z, axis=-1, keepdims=True)


def _gemm_body(be_ref, xg_ref, w1_ref, b1_ref, w2_ref, b2_ref, og_ref):
    del be_ref
    xb = xg_ref[...].astype(jnp.bfloat16)
    h = jnp.maximum(
        jnp.dot(xb, w1_ref[0], preferred_element_type=jnp.float32) + b1_ref[0],
        0.0)
    og_ref[...] = jnp.dot(h.astype(jnp.bfloat16), w2_ref[0],
                          preferred_element_type=jnp.float32) + b2_ref[0]


def _make_scatter(n, d, pt, ch):
    b_per_w = n // _NW
    nch = b_per_w // ch
    mesh = plsc.VectorSubcoreMesh(core_axis_name="c", subcore_axis_name="s")

    @functools.partial(
        pl.kernel, mesh=mesh,
        out_type=jax.ShapeDtypeStruct((pt, d), jnp.float32),
        scratch_types=[
            pltpu.VMEM((ch,), jnp.int32),
            pltpu.VMEM((ch,), jnp.int32),
            pltpu.VMEM((ch, d), jnp.float32),
            pltpu.SemaphoreType.DMA,
        ],
    )
    def k(x_hbm, p0_hbm, p1_hbm, xg_hbm, i0_v, i1_v, rows_v, sem):
        wid = lax.axis_index("s") * _NC + lax.axis_index("c")
        base = wid * b_per_w

        def body(c, carry):
            cb = pl.multiple_of(base + c * ch, 8)
            pltpu.sync_copy(p0_hbm.at[pl.ds(cb, ch)], i0_v)
            pltpu.sync_copy(p1_hbm.at[pl.ds(cb, ch)], i1_v)
            pltpu.sync_copy(x_hbm.at[pl.ds(cb, ch)], rows_v)
            cp0 = pltpu.async_copy(rows_v, xg_hbm.at[i0_v], sem)
            cp1 = pltpu.async_copy(rows_v, xg_hbm.at[i1_v], sem)
            cp0.wait()
            cp1.wait()
            return carry

        lax.fori_loop(0, nch, body, 0)

    return k


def _make_combine(pt, d, n, ch):
    b_per_w = n // _NW
    nch = b_per_w // ch
    nlane = d // 16
    mesh = plsc.VectorSubcoreMesh(core_axis_name="c", subcore_axis_name="s")

    @functools.partial(
        pl.kernel, mesh=mesh,
        out_type=jax.ShapeDtypeStruct((n, d), jnp.float32),
        scratch_types=[
            pltpu.VMEM((ch,), jnp.int32),
            pltpu.VMEM((ch,), jnp.int32),
            pltpu.VMEM((ch, 16), jnp.float32),
            pltpu.VMEM((ch, 16), jnp.float32),
            pltpu.VMEM((ch, d), jnp.float32),
            pltpu.VMEM((ch, d), jnp.float32),
            pltpu.SemaphoreType.DMA,
        ],
    )
    def k(og_hbm, p0_hbm, p1_hbm, g0_hbm, g1_hbm, out_hbm,
          i0_v, i1_v, g0_v, g1_v, r0_v, r1_v, sem):
        wid = lax.axis_index("s") * _NC + lax.axis_index("c")
        base = wid * b_per_w

        def body(c, carry):
            cb = pl.multiple_of(base + c * ch, 8)
            pltpu.sync_copy(p0_hbm.at[pl.ds(cb, ch)], i0_v)
            pltpu.sync_copy(p1_hbm.at[pl.ds(cb, ch)], i1_v)
            pltpu.sync_copy(g0_hbm.at[pl.ds(cb, ch)], g0_v)
            pltpu.sync_copy(g1_hbm.at[pl.ds(cb, ch)], g1_v)
            cp0 = pltpu.async_copy(og_hbm.at[i0_v], r0_v, sem)
            cp1 = pltpu.async_copy(og_hbm.at[i1_v], r1_v, sem)
            cp0.wait()
            cp1.wait()

            def addrow(i, c2):
                ga = g0_v[i, :]
                gb = g1_v[i, :]
                for j in range(nlane):
                    sl = pl.ds(j * 16, 16)
                    r0_v[i, sl] = r0_v[i, sl] * ga + r1_v[i, sl] * gb
                return c2

            lax.fori_loop(0, ch, addrow, 0)
            pltpu.sync_copy(r0_v, out_hbm.at[pl.ds(cb, ch)])
            return carry

        lax.fori_loop(0, nch, body, 0)

    return k


def kernel(x, Wg, bg, Wn, bn, W1, b1, W2, b2, eps):
    B, S, D = x.shape
    E = Wg.shape[1]
    FF = W1.shape[2]
    N = B * S
    P = 2 * N
    NB = P // _BT + E
    PT = NB * _BT
    x2 = x.reshape(N, D)
    eps2 = eps.reshape(N, E)

    bt_r = min(2048, N)
    gates = pl.pallas_call(
        _router_body,
        grid=(N // bt_r,),
        in_specs=[
            pl.BlockSpec((bt_r, D), lambda t: (t, 0)),
            pl.BlockSpec((D, E), lambda t: (0, 0)),
            pl.BlockSpec((1, E), lambda t: (0, 0)),
            pl.BlockSpec((D, E), lambda t: (0, 0)),
            pl.BlockSpec((1, E), lambda t: (0, 0)),
            pl.BlockSpec((bt_r, E), lambda t: (t, 0)),
        ],
        out_specs=pl.BlockSpec((bt_r, E), lambda t: (t, 0)),
        out_shape=jax.ShapeDtypeStruct((N, E), jnp.float32),
        compiler_params=pltpu.CompilerParams(
            dimension_semantics=("arbitrary",)),
    )(x2, Wg, bg.reshape(1, E), Wn, bn.reshape(1, E), eps2)

    mask = gates > 0.0
    mi = mask.astype(jnp.int32)
    cnt = jnp.sum(mi, axis=0)
    rank = jnp.cumsum(mi, axis=0) - mi
    padded = ((cnt + _BT - 1) // _BT) * _BT
    pad_end = jnp.cumsum(padded)
    pad_off = pad_end - padded
    destm = pad_off[None, :] + rank
    posm = jnp.where(mask, destm, PT)
    pos0 = jnp.min(posm, axis=1).astype(jnp.int32)
    pos1 = jnp.min(jnp.where(posm == pos0[:, None], PT, posm),
                   axis=1).astype(jnp.int32)
    ii = jax.lax.broadcasted_iota(jnp.int32, (N, E), 1)
    g0 = jnp.sum(jnp.where(posm == pos0[:, None], gates, 0.0), axis=1)
    g1 = jnp.sum(jnp.where(posm == pos1[:, None], gates, 0.0), axis=1)
    del ii
    g0b = jnp.broadcast_to(g0[:, None], (N, 16))
    g1b = jnp.broadcast_to(g1[:, None], (N, 16))
    block_expert = jnp.minimum(
        jnp.searchsorted(pad_end, jnp.arange(NB, dtype=jnp.int32) * _BT,
                         side="right").astype(jnp.int32), E - 1)

    xg = _make_scatter(N, D, PT, 64)(x2, pos0, pos1)

    og = pl.pallas_call(
        _gemm_body,
        grid_spec=pltpu.PrefetchScalarGridSpec(
            num_scalar_prefetch=1,
            grid=(NB,),
            in_specs=[
                pl.BlockSpec((_BT, D), lambda b, be: (b, 0)),
                pl.BlockSpec((1, D, FF), lambda b, be: (be[b], 0, 0)),
                pl.BlockSpec((1, 1, FF), lambda b, be: (be[b], 0, 0)),
                pl.BlockSpec((1, FF, D), lambda b, be: (be[b], 0, 0)),
                pl.BlockSpec((1, 1, D), lambda b, be: (be[b], 0, 0)),
            ],
            out_specs=pl.BlockSpec((_BT, D), lambda b, be: (b, 0)),
        ),
        out_shape=jax.ShapeDtypeStruct((PT, D), jnp.float32),
        compiler_params=pltpu.CompilerParams(
            dimension_semantics=("arbitrary",),
            vmem_limit_bytes=100 * 1024 * 1024),
    )(block_expert, xg, W1.astype(jnp.bfloat16), b1.reshape(E, 1, FF),
      W2.astype(jnp.bfloat16), b2.reshape(E, 1, D))

    out = _make_combine(PT, D, N, 32)(og, pos0, pos1, g0b, g1b)
    return out.reshape(B, S, D)


# speedup vs baseline: 1.0602x; 1.0602x over previous
import functools

import jax
import jax.numpy as jnp
from jax import lax
from jax.experimental import pallas as pl
from jax.experimental.pallas import tpu as pltpu
from jax.experimental.pallas import tpu_sc as plsc

_BT = 256
_NC = 2
_NS = 16
_NW = _NC * _NS


def _router_body(x_ref, wg_ref, bg_ref, wn_ref, bn_ref, eps_ref, g_ref):
    xb = x_ref[...]
    lg = jnp.dot(xb, wg_ref[...], preferred_element_type=jnp.float32) + bg_ref[...]
    nz = jnp.dot(xb, wn_ref[...], preferred_element_type=jnp.float32) + bn_ref[...]
    sp = jnp.maximum(nz, 0.0) + jnp.log1p(jnp.exp(-jnp.abs(nz)))
    nl = lg + eps_ref[...] * sp
    e = nl.shape[-1]
    m1 = jnp.max(nl, axis=-1, keepdims=True)
    ii = jax.lax.broadcasted_iota(jnp.int32, nl.shape, 1)
    fmi = jnp.min(jnp.where(nl == m1, ii, e), axis=-1, keepdims=True)
    m2 = jnp.max(jnp.where(ii == fmi, -jnp.inf, nl), axis=-1, keepdims=True)
    sel = (ii == fmi) | (nl >= m2)
    z = jnp.where(sel, jnp.exp(nl - m1), 0.0)
    g_ref[...] = z / jnp.sum(z, axis=-1, keepdims=True)


def _make_dispatch_body(bt, pt, nb, wbe):
    def body(g_ref, pos01_ref, g0b_ref, g1b_ref, be_ref):
        g = g_ref[...]
        n, e = g.shape
        mask = g > 0.0
        mi = mask.astype(jnp.int32)
        c = mi
        k = 1
        while k < n:
            sh = jnp.concatenate(
                [jnp.zeros((k, e), jnp.int32), c[:-k]], axis=0)
            c = c + sh
            k *= 2
        cnt = c[n - 1:n, :]
        rank = c - mi
        padded = ((cnt + bt - 1) // bt) * bt
        pe = padded
        k = 1
        while k < e:
            pe = pe + jnp.concatenate(
                [jnp.zeros((1, k), jnp.int32), pe[:, :-k]], axis=1)
            k *= 2
        pad_off = pe - padded
        destm = pad_off + rank
        posm = jnp.where(mask, destm, pt)
        pos0 = jnp.min(posm, axis=1, keepdims=True)
        pos1 = jnp.min(jnp.where(posm == pos0, pt, posm), axis=1,
                       keepdims=True)
        g0 = jnp.sum(jnp.where(posm == pos0, g, 0.0), axis=1, keepdims=True)
        g1 = jnp.sum(jnp.where(posm == pos1, g, 0.0), axis=1, keepdims=True)
        ii = jax.lax.broadcasted_iota(jnp.int32, (n, e), 1)
        pos01_ref[...] = jnp.where(ii == 0, pos0,
                                   jnp.where(ii == 1, pos1, 0))
        g0b_ref[...] = jnp.broadcast_to(g0, (n, 16))
        g1b_ref[...] = jnp.broadcast_to(g1, (n, 16))
        bidx = jax.lax.broadcasted_iota(jnp.int32, (1, wbe), 1) * bt
        acc = jnp.zeros((1, wbe), jnp.int32)
        for j in range(e):
            acc = acc + (pe[:, j:j + 1] <= bidx).astype(jnp.int32)
        be_ref[...] = jnp.minimum(acc, e - 1)
    return body


def _gemm_body(be_ref, xg_ref, w1_ref, b1_ref, w2_ref, b2_ref, og_ref):
    del be_ref
    xb = xg_ref[...].astype(jnp.bfloat16)
    h = jnp.maximum(
        jnp.dot(xb, w1_ref[0], preferred_element_type=jnp.float32) + b1_ref[0],
        0.0)
    og_ref[...] = jnp.dot(h.astype(jnp.bfloat16), w2_ref[0],
                          preferred_element_type=jnp.float32) + b2_ref[0]


def _make_scatter(n, d, pt, ch):
    b_per_w = n // _NW
    nch = b_per_w // ch
    mesh = plsc.VectorSubcoreMesh(core_axis_name="c", subcore_axis_name="s")

    @functools.partial(
        pl.kernel, mesh=mesh,
        out_type=jax.ShapeDtypeStruct((pt, d), jnp.float32),
        scratch_types=[
            pltpu.VMEM((ch,), jnp.int32),
            pltpu.VMEM((ch,), jnp.int32),
            pltpu.VMEM((ch, d), jnp.float32),
            pltpu.SemaphoreType.DMA,
        ],
    )
    def k(x_hbm, p0_hbm, p1_hbm, xg_hbm, i0_v, i1_v, rows_v, sem):
        wid = lax.axis_index("s") * _NC + lax.axis_index("c")
        base = wid * b_per_w

        def body(c, carry):
            cb = pl.multiple_of(base + c * ch, 8)
            pltpu.sync_copy(p0_hbm.at[pl.ds(cb, ch)], i0_v)
            pltpu.sync_copy(p1_hbm.at[pl.ds(cb, ch)], i1_v)
            pltpu.sync_copy(x_hbm.at[pl.ds(cb, ch)], rows_v)
            cp0 = pltpu.async_copy(rows_v, xg_hbm.at[i0_v], sem)
            cp1 = pltpu.async_copy(rows_v, xg_hbm.at[i1_v], sem)
            cp0.wait()
            cp1.wait()
            return carry

        lax.fori_loop(0, nch, body, 0)

    return k


def _make_combine(pt, d, n, ch):
    b_per_w = n // _NW
    nch = b_per_w // ch
    nlane = d // 16
    mesh = plsc.VectorSubcoreMesh(core_axis_name="c", subcore_axis_name="s")

    @functools.partial(
        pl.kernel, mesh=mesh,
        out_type=jax.ShapeDtypeStruct((n, d), jnp.float32),
        scratch_types=[
            pltpu.VMEM((ch,), jnp.int32),
            pltpu.VMEM((ch,), jnp.int32),
            pltpu.VMEM((ch, 16), jnp.float32),
            pltpu.VMEM((ch, 16), jnp.float32),
            pltpu.VMEM((ch, d), jnp.float32),
            pltpu.VMEM((ch, d), jnp.float32),
            pltpu.SemaphoreType.DMA,
        ],
    )
    def k(og_hbm, p0_hbm, p1_hbm, g0_hbm, g1_hbm, out_hbm,
          i0_v, i1_v, g0_v, g1_v, r0_v, r1_v, sem):
        wid = lax.axis_index("s") * _NC + lax.axis_index("c")
        base = wid * b_per_w

        def body(c, carry):
            cb = pl.multiple_of(base + c * ch, 8)
            pltpu.sync_copy(p0_hbm.at[pl.ds(cb, ch)], i0_v)
            pltpu.sync_copy(p1_hbm.at[pl.ds(cb, ch)], i1_v)
            pltpu.sync_copy(g0_hbm.at[pl.ds(cb, ch)], g0_v)
            pltpu.sync_copy(g1_hbm.at[pl.ds(cb, ch)], g1_v)
            cp0 = pltpu.async_copy(og_hbm.at[i0_v], r0_v, sem)
            cp1 = pltpu.async_copy(og_hbm.at[i1_v], r1_v, sem)
            cp0.wait()
            cp1.wait()

            def addrow(i, c2):
                ga = g0_v[i, :]
                gb = g1_v[i, :]
                for j in range(nlane):
                    sl = pl.ds(j * 16, 16)
                    r0_v[i, sl] = r0_v[i, sl] * ga + r1_v[i, sl] * gb
                return c2

            lax.fori_loop(0, ch, addrow, 0)
            pltpu.sync_copy(r0_v, out_hbm.at[pl.ds(cb, ch)])
            return carry

        lax.fori_loop(0, nch, body, 0)

    return k


def kernel(x, Wg, bg, Wn, bn, W1, b1, W2, b2, eps):
    B, S, D = x.shape
    E = Wg.shape[1]
    FF = W1.shape[2]
    N = B * S
    P = 2 * N
    NB = P // _BT + E
    PT = NB * _BT
    x2 = x.reshape(N, D)
    eps2 = eps.reshape(N, E)

    bt_r = min(2048, N)
    gates = pl.pallas_call(
        _router_body,
        grid=(N // bt_r,),
        in_specs=[
            pl.BlockSpec((bt_r, D), lambda t: (t, 0)),
            pl.BlockSpec((D, E), lambda t: (0, 0)),
            pl.BlockSpec((1, E), lambda t: (0, 0)),
            pl.BlockSpec((D, E), lambda t: (0, 0)),
            pl.BlockSpec((1, E), lambda t: (0, 0)),
            pl.BlockSpec((bt_r, E), lambda t: (t, 0)),
        ],
        out_specs=pl.BlockSpec((bt_r, E), lambda t: (t, 0)),
        out_shape=jax.ShapeDtypeStruct((N, E), jnp.float32),
        compiler_params=pltpu.CompilerParams(
            dimension_semantics=("arbitrary",)),
    )(x2, Wg, bg.reshape(1, E), Wn, bn.reshape(1, E), eps2)

    WBE = ((NB + 127) // 128) * 128
    pos01, g0b, g1b, be_mat = pl.pallas_call(
        _make_dispatch_body(_BT, PT, NB, WBE),
        out_shape=(
            jax.ShapeDtypeStruct((N, E), jnp.int32),
            jax.ShapeDtypeStruct((N, 16), jnp.float32),
            jax.ShapeDtypeStruct((N, 16), jnp.float32),
            jax.ShapeDtypeStruct((1, WBE), jnp.int32),
        ),
    )(gates)
    pos0 = pos01[:, 0]
    pos1 = pos01[:, 1]
    block_expert = be_mat[0, :NB]

    xg = _make_scatter(N, D, PT, 64)(x2, pos0, pos1)

    og = pl.pallas_call(
        _gemm_body,
        grid_spec=pltpu.PrefetchScalarGridSpec(
            num_scalar_prefetch=1,
            grid=(NB,),
            in_specs=[
                pl.BlockSpec((_BT, D), lambda b, be: (b, 0)),
                pl.BlockSpec((1, D, FF), lambda b, be: (be[b], 0, 0)),
                pl.BlockSpec((1, 1, FF), lambda b, be: (be[b], 0, 0)),
                pl.BlockSpec((1, FF, D), lambda b, be: (be[b], 0, 0)),
                pl.BlockSpec((1, 1, D), lambda b, be: (be[b], 0, 0)),
            ],
            out_specs=pl.BlockSpec((_BT, D), lambda b, be: (b, 0)),
        ),
        out_shape=jax.ShapeDtypeStruct((PT, D), jnp.float32),
        compiler_params=pltpu.CompilerParams(
            dimension_semantics=("arbitrary",),
            vmem_limit_bytes=100 * 1024 * 1024),
    )(block_expert, xg, W1.astype(jnp.bfloat16), b1.reshape(E, 1, FF),
      W2.astype(jnp.bfloat16), b2.reshape(E, 1, D))

    out = _make_combine(PT, D, N, 32)(og, pos0, pos1, g0b, g1b)
    return out.reshape(B, S, D)
